# baseline (speedup 1.0000x reference)
import functools

import jax
import jax.numpy as jnp
import numpy as np
from jax import lax
from jax.experimental import pallas as pl
from jax.experimental.pallas import tpu as pltpu
import jax.experimental.pallas.tpu_sc as plsc

D = 1024
H = 16
DH = D // H
FF = 2048
E = 8
K = 2

BM = 256
BN = 512
BQ = 256
BLK = 256
NJ = 4
FFC = FF // NJ
CH = 32
NWORK = 32


def _mm_bias_kernel(x_ref, w_ref, b_ref, o_ref):
    o_ref[...] = (
        jnp.dot(x_ref[...], w_ref[...], preferred_element_type=jnp.float32)
        + b_ref[...]
    )


def _mm_bias(x, w, b):
    m, d = x.shape
    n = w.shape[1]
    grid = (n // BN, m // BM)
    return pl.pallas_call(
        _mm_bias_kernel,
        grid=grid,
        in_specs=[
            pl.BlockSpec((BM, d), lambda j, i: (i, 0)),
            pl.BlockSpec((d, BN), lambda j, i: (0, j)),
            pl.BlockSpec((1, BN), lambda j, i: (0, j)),
        ],
        out_specs=pl.BlockSpec((BM, BN), lambda j, i: (i, j)),
        out_shape=jax.ShapeDtypeStruct((m, n), jnp.float32),
    )(x, w, b.reshape(1, n))


def _mm_res_ln_kernel(x_ref, w_ref, b_ref, r_ref, g_ref, bb_ref, o_ref):
    y = (
        jnp.dot(x_ref[...], w_ref[...], preferred_element_type=jnp.float32)
        + b_ref[...]
        + r_ref[...]
    )
    mu = jnp.mean(y, axis=-1, keepdims=True)
    var = jnp.mean((y - mu) ** 2, axis=-1, keepdims=True)
    o_ref[...] = (y - mu) * lax.rsqrt(var + 1e-5) * g_ref[...] + bb_ref[...]


def _mm_res_ln(x, w, b, res, g, beta):
    m, d = x.shape
    return pl.pallas_call(
        _mm_res_ln_kernel,
        grid=(m // BM,),
        in_specs=[
            pl.BlockSpec((BM, d), lambda i: (i, 0)),
            pl.BlockSpec((d, d), lambda i: (0, 0)),
            pl.BlockSpec((1, d), lambda i: (0, 0)),
            pl.BlockSpec((BM, d), lambda i: (i, 0)),
            pl.BlockSpec((1, d), lambda i: (0, 0)),
            pl.BlockSpec((1, d), lambda i: (0, 0)),
        ],
        out_specs=pl.BlockSpec((BM, d), lambda i: (i, 0)),
        out_shape=jax.ShapeDtypeStruct((m, d), jnp.float32),
    )(x, w, b.reshape(1, d), res, g.reshape(1, d), beta.reshape(1, d))


def _attn_kernel(q_ref, k_ref, v_ref, o_ref):
    q = q_ref[0]
    s = lax.dot_general(
        q, k_ref[0], (((1,), (1,)), ((), ())),
        preferred_element_type=jnp.float32,
    ) * (1.0 / np.sqrt(DH).astype(np.float32))
    m = jnp.max(s, axis=-1, keepdims=True)
    p = jnp.exp(s - m)
    den = jnp.sum(p, axis=-1, keepdims=True)
    o = jnp.dot(p, v_ref[0], preferred_element_type=jnp.float32)
    o_ref[0] = o / den


def _attn(q, k, v):
    h, sq, dh = q.shape
    sk = k.shape[1]
    return pl.pallas_call(
        _attn_kernel,
        grid=(h, sq // BQ),
        in_specs=[
            pl.BlockSpec((1, BQ, dh), lambda hh, i: (hh, i, 0)),
            pl.BlockSpec((1, sk, dh), lambda hh, i: (hh, 0, 0)),
            pl.BlockSpec((1, sk, dh), lambda hh, i: (hh, 0, 0)),
        ],
        out_specs=pl.BlockSpec((1, BQ, dh), lambda hh, i: (hh, i, 0)),
        out_shape=jax.ShapeDtypeStruct((h, sq, dh), jnp.float32),
    )(q, k, v)


def _router_kernel(x_ref, wr_ref, br_ref, g_ref):
    logit = (
        jnp.dot(x_ref[...], wr_ref[...], preferred_element_type=jnp.float32)
        + br_ref[...]
    )
    ei = lax.broadcasted_iota(jnp.int32, logit.shape, 1)
    m1 = jnp.max(logit, axis=-1, keepdims=True)
    i1 = jnp.min(jnp.where(logit == m1, ei, E), axis=-1, keepdims=True)
    lm = jnp.where(ei == i1, -jnp.inf, logit)
    m2 = jnp.max(lm, axis=-1, keepdims=True)
    i2 = jnp.min(jnp.where(lm == m2, ei, E), axis=-1, keepdims=True)
    w1 = 1.0 / (1.0 + jnp.exp(m2 - m1))
    w2 = 1.0 - w1
    g_ref[...] = jnp.where(ei == i1, w1, 0.0) + jnp.where(ei == i2, w2, 0.0)


def _router(x, wr, br):
    m = x.shape[0]
    return pl.pallas_call(
        _router_kernel,
        grid=(m // BM,),
        in_specs=[
            pl.BlockSpec((BM, D), lambda i: (i, 0)),
            pl.BlockSpec((D, E), lambda i: (0, 0)),
            pl.BlockSpec((1, E), lambda i: (0, 0)),
        ],
        out_specs=pl.BlockSpec((BM, E), lambda i: (i, 0)),
        out_shape=jax.ShapeDtypeStruct((m, E), jnp.float32),
    )(x, wr, br.reshape(1, E))


def _ffn_kernel(eid_ref, x_ref, w1_ref, b1_ref, w2_ref, b2_ref, o_ref):
    del eid_ref
    j = pl.program_id(1)
    h = (
        jnp.dot(x_ref[...], w1_ref[0], preferred_element_type=jnp.float32)
        + b1_ref[0]
    )
    h = 0.5 * h * (1.0 + lax.erf(h * np.float32(0.7071067811865476)))
    y = jnp.dot(h, w2_ref[0], preferred_element_type=jnp.float32)

    @pl.when(j == 0)
    def _():
        o_ref[...] = y + b2_ref[0]

    @pl.when(j > 0)
    def _():
        o_ref[...] += y


def _moe_ffn(xp, w1, b1, w2, b2, eid):
    p = xp.shape[0]
    nb = p // BLK
    grid_spec = pltpu.PrefetchScalarGridSpec(
        num_scalar_prefetch=1,
        grid=(nb, NJ),
        in_specs=[
            pl.BlockSpec((BLK, D), lambda b, j, eid: (b, 0)),
            pl.BlockSpec((1, D, FFC), lambda b, j, eid: (eid[b], 0, j)),
            pl.BlockSpec((1, 1, FFC), lambda b, j, eid: (eid[b], 0, j)),
            pl.BlockSpec((1, FFC, D), lambda b, j, eid: (eid[b], j, 0)),
            pl.BlockSpec((1, 1, D), lambda b, j, eid: (eid[b], 0, 0)),
        ],
        out_specs=pl.BlockSpec((BLK, D), lambda b, j, eid: (b, 0)),
    )
    return pl.pallas_call(
        _ffn_kernel,
        grid_spec=grid_spec,
        out_shape=jax.ShapeDtypeStruct((p, D), jnp.float32),
    )(eid, xp, w1, b1.reshape(E, 1, FF), w2, b2.reshape(E, 1, D))


def _combine_ln_kernel(x_ref, ya_ref, yb_ref, wa_ref, wb_ref, g_ref, bb_ref,
                       o_ref):
    y = x_ref[...] + wa_ref[...] * ya_ref[...] + wb_ref[...] * yb_ref[...]
    mu = jnp.mean(y, axis=-1, keepdims=True)
    var = jnp.mean((y - mu) ** 2, axis=-1, keepdims=True)
    o_ref[...] = (y - mu) * lax.rsqrt(var + 1e-5) * g_ref[...] + bb_ref[...]


def _combine_ln(x, ya, yb, wa, wb, g, beta):
    m = x.shape[0]
    return pl.pallas_call(
        _combine_ln_kernel,
        grid=(m // BM,),
        in_specs=[
            pl.BlockSpec((BM, D), lambda i: (i, 0)),
            pl.BlockSpec((BM, D), lambda i: (i, 0)),
            pl.BlockSpec((BM, D), lambda i: (i, 0)),
            pl.BlockSpec((BM, 1), lambda i: (i, 0)),
            pl.BlockSpec((BM, 1), lambda i: (i, 0)),
            pl.BlockSpec((1, D), lambda i: (0, 0)),
            pl.BlockSpec((1, D), lambda i: (0, 0)),
        ],
        out_specs=pl.BlockSpec((BM, D), lambda i: (i, 0)),
        out_shape=jax.ShapeDtypeStruct((m, D), jnp.float32),
    )(x, ya, yb, wa, wb, g.reshape(1, D), beta.reshape(1, D))


@functools.lru_cache(maxsize=None)
def _sc_gather_fn(v_rows, r_rows):
    rows_w = r_rows // NWORK
    nch = rows_w // CH
    mesh = plsc.VectorSubcoreMesh(core_axis_name="c", subcore_axis_name="s")

    @functools.partial(
        pl.kernel,
        mesh=mesh,
        out_type=jax.ShapeDtypeStruct((r_rows, D), jnp.float32),
        scratch_types=[
            pltpu.VMEM((rows_w,), jnp.int32),
            pltpu.VMEM((CH, D), jnp.float32),
            pltpu.VMEM((CH, D), jnp.float32),
            pltpu.SemaphoreType.DMA,
            pltpu.SemaphoreType.DMA,
        ],
    )
    def gather(table_hbm, idx_hbm, out_hbm, idx_v, buf0, buf1, sem0, sem1):
        wid = lax.axis_index("s") * 2 + lax.axis_index("c")
        base = wid * rows_w
        pltpu.sync_copy(idx_hbm.at[pl.ds(base, rows_w)], idx_v)
        bufs = (buf0, buf1)
        sems = (sem0, sem1)
        cps = [None, None]
        cps[0] = pltpu.async_copy(
            table_hbm.at[idx_v.at[pl.ds(0, CH)]], bufs[0], sems[0])
        for c in range(nch):
            nxt = c + 1
            if nxt < nch:
                cps[nxt % 2] = pltpu.async_copy(
                    table_hbm.at[idx_v.at[pl.ds(nxt * CH, CH)]],
                    bufs[nxt % 2], sems[nxt % 2])
            cps[c % 2].wait()
            pltpu.sync_copy(bufs[c % 2],
                            out_hbm.at[pl.ds(base + c * CH, CH)])

    return gather


def _gather_rows(table, idx):
    return _sc_gather_fn(table.shape[0], idx.shape[0])(table, idx)


def _moe_layer(x, wr, br, w1, b1, w2, b2, g3, b3):
    s = x.shape[0]
    nassign = s * K
    nb = nassign // BLK + E - 1
    nb = ((nb * BLK // (NWORK * CH)) + 1) * (NWORK * CH) // BLK
    p = nb * BLK

    gates = _router(x, wr, br)
    topw, topi = lax.top_k(gates, K)

    ee = topi.reshape(-1).astype(jnp.int32)
    perm = jnp.argsort(ee)
    sorted_e = ee[perm]
    counts = jnp.sum(
        (ee[:, None] == jnp.arange(E, dtype=jnp.int32)[None, :]).astype(
            jnp.int32), axis=0)
    nb_e = (counts + BLK - 1) // BLK
    cum_blocks = jnp.cumsum(nb_e)
    bstart = (cum_blocks - nb_e) * BLK
    gstart = jnp.cumsum(counts) - counts
    rank = jnp.arange(nassign, dtype=jnp.int32) - gstart[sorted_e]
    dst = bstart[sorted_e] + rank
    tok = (perm // K).astype(jnp.int32)
    src_idx = jnp.zeros(p, jnp.int32).at[dst].set(tok)
    pos = jnp.zeros(nassign, jnp.int32).at[perm].set(dst)
    pos = pos.reshape(s, K)
    eid = jnp.minimum(
        jnp.searchsorted(cum_blocks, jnp.arange(nb, dtype=jnp.int32),
                         side="right").astype(jnp.int32), E - 1)

    xp = _gather_rows(x, src_idx)
    yp = _moe_ffn(xp, w1, b1, w2, b2, eid)
    yab = _gather_rows(yp, pos.T.reshape(-1))
    ya, yb = yab[:s], yab[s:]
    return _combine_ln(x, ya, yb, topw[:, :1], topw[:, 1:], g3, b3)


def _split_heads(x):
    s = x.shape[0]
    return x.reshape(s, H, DH).transpose(1, 0, 2)


def _attn_block(xq, xkv, wq, bq, wkv, bkv, wo, bo, res, g, beta):
    q = _split_heads(_mm_bias(xq, wq, bq))
    kv = _mm_bias(xkv, wkv, bkv)
    k = _split_heads(kv[:, :D])
    v = _split_heads(kv[:, D:])
    o = _attn(q, k, v)
    o = o.transpose(1, 0, 2).reshape(xq.shape[0], D)
    return _mm_res_ln(o, wo, bo, res, g, beta)


def kernel(tgt, memory, sa_Wq, sa_bq, sa_Wk, sa_bk, sa_Wv, sa_bv, sa_Wo, sa_bo,
           ca_Wq, ca_bq, ca_Wk, ca_bk, ca_Wv, ca_bv, ca_Wo, ca_bo,
           Wr, br, W1, b1, W2, b2,
           ln1_g, ln1_b, ln2_g, ln2_b, ln3_g, ln3_b):
    x = tgt[0]
    mem = memory[0]
    nlayers = sa_Wq.shape[0]
    for l in range(nlayers):
        skv_w = jnp.concatenate([sa_Wk[l], sa_Wv[l]], axis=1)
        skv_b = jnp.concatenate([sa_bk[l], sa_bv[l]])
        x = _attn_block(x, x, sa_Wq[l], sa_bq[l], skv_w, skv_b,
                        sa_Wo[l], sa_bo[l], x, ln1_g[l], ln1_b[l])
        ckv_w = jnp.concatenate([ca_Wk[l], ca_Wv[l]], axis=1)
        ckv_b = jnp.concatenate([ca_bk[l], ca_bv[l]])
        x = _attn_block(x, mem, ca_Wq[l], ca_bq[l], ckv_w, ckv_b,
                        ca_Wo[l], ca_bo[l], x, ln2_g[l], ln2_b[l])
        x = _moe_layer(x, Wr[l], br[l], W1[l], b1[l], W2[l], b2[l],
                       ln3_g[l], ln3_b[l])
    return x[None]
